# baseline (speedup 1.0000x reference)
import functools

import jax
import jax.numpy as jnp
from jax import lax
from jax.experimental import pallas as pl
from jax.experimental.pallas import tpu as pltpu
from jax.experimental.pallas import tpu_sc as plsc

_VOCAB = 100000
_NUM_CAT = 26
_EMB_DIM = 32
_LAMBDA = 0.001

_CHUNK = 4096
_LANES = 16

_ROWS = (_NUM_CAT * _VOCAB * _EMB_DIM) // 128
_RB = 5000
_GRID = _ROWS // _RB


def _sc_hist_body(ids_hbm, hist_hbm, idbuf, hist_v, sem0, sem1):
    n_ids = ids_hbm.shape[0] // _NUM_CAT
    nch = n_ids // _CHUNK
    wid = lax.axis_index("s") * 2 + lax.axis_index("c")

    @pl.when(wid < _NUM_CAT)
    def _():
        def zero_body(j, carry):
            hist_v[pl.ds(j * _LANES, _LANES)] = jnp.zeros((_LANES,), jnp.int32)
            return carry

        lax.fori_loop(0, _VOCAB // _LANES, zero_body, 0, unroll=16)

        sems = (sem0, sem1)

        def dma(c, b):
            base = wid * n_ids + c * _CHUNK
            return pltpu.make_async_copy(
                ids_hbm.at[pl.ds(base, _CHUNK)], idbuf.at[b], sems[b]
            )

        def process(b):
            def vec_body(j, carry):
                v = idbuf[b, pl.ds(j * _LANES, _LANES)]
                cnt, last = plsc.scan_count(v)
                plsc.addupdate_scatter(hist_v, [v], cnt, mask=last)
                return carry

            lax.fori_loop(0, _CHUNK // _LANES, vec_body, 0, unroll=8)

        dma(0, 0).start()

        def outer(k, carry):
            c0 = 2 * k
            dma(c0 + 1, 1).start()
            dma(c0, 0).wait()
            process(0)
            dma(jnp.minimum(c0 + 2, nch - 1), 0).start()
            dma(c0 + 1, 1).wait()
            process(1)
            return carry

        lax.fori_loop(0, nch // 2, outer, 0)
        dma(nch - 1, 0).wait()

        pltpu.sync_copy(hist_v, hist_hbm.at[pl.ds(wid * _VOCAB, _VOCAB)])


def _sc_hist(ids_flat):
    mesh = plsc.VectorSubcoreMesh(
        core_axis_name="c", subcore_axis_name="s", num_cores=2, num_subcores=16
    )
    return pl.kernel(
        _sc_hist_body,
        out_type=jax.ShapeDtypeStruct((_NUM_CAT * _VOCAB,), jnp.int32),
        mesh=mesh,
        scratch_types=[
            pltpu.VMEM((2, _CHUNK), jnp.int32),
            pltpu.VMEM((_VOCAB,), jnp.int32),
            pltpu.SemaphoreType.DMA,
            pltpu.SemaphoreType.DMA,
        ],
        compiler_params=pltpu.CompilerParams(needs_layout_passes=False),
    )(ids_flat)


def _tc_reduce_body(tab_ref, hist_ref, out_ref, *, n_total):
    g = pl.program_id(0)

    @pl.when(g == 0)
    def _():
        out_ref[0, 0] = 0.0

    x = tab_ref[...]
    x2 = x * x
    grp = (
        lax.broadcasted_iota(jnp.int32, (128, 4), 0) // _EMB_DIM
        == lax.broadcasted_iota(jnp.int32, (128, 4), 1)
    ).astype(jnp.float32)
    w2 = jax.lax.dot_general(
        x2, grp, (((1,), (0,)), ((), ())), preferred_element_type=jnp.float32
    )
    h = hist_ref[...].astype(jnp.float32)
    inv_sqrt = lax.rsqrt(h / n_total + 1e-9)
    out_ref[0, 0] += jnp.sum(inv_sqrt * w2)


def _tc_reduce(tab2, hist4, n_total):
    body = functools.partial(_tc_reduce_body, n_total=float(n_total))
    return pl.pallas_call(
        body,
        grid=(_GRID,),
        in_specs=[
            pl.BlockSpec((_RB, 128), lambda g: (g, 0)),
            pl.BlockSpec((_RB, 4), lambda g: (g, 0)),
        ],
        out_specs=pl.BlockSpec(memory_space=pltpu.SMEM),
        out_shape=jax.ShapeDtypeStruct((1, 1), jnp.float32),
    )(tab2, hist4)


def kernel(inputs, tables):
    b, t, c = inputs.shape
    n_total = b * t
    ids = (
        inputs[:, :, c - _NUM_CAT :]
        .astype(jnp.int32)
        .reshape(n_total, _NUM_CAT)
        .T.reshape(-1)
    )
    hist = _sc_hist(ids)
    tab2 = tables.reshape(_ROWS, 128)
    hist4 = hist.reshape(_ROWS, 4)
    total = _tc_reduce(tab2, hist4, n_total)
    loss = (_LAMBDA / _VOCAB) * total[0, 0]
    return (inputs, loss)


# speedup vs baseline: 1.1717x; 1.1717x over previous
import functools

import jax
import jax.numpy as jnp
from jax import lax
from jax.experimental import pallas as pl
from jax.experimental.pallas import tpu as pltpu
from jax.experimental.pallas import tpu_sc as plsc

_VOCAB = 100000
_NUM_CAT = 26
_EMB_DIM = 32
_LAMBDA = 0.001

_CHUNK = 4096
_LANES = 16

_BV = 5000
_NB = _VOCAB // _BV
_GRID = _NUM_CAT * _NB


def _sc_hist_body(ids_hbm, hist_hbm, idbuf, hist_v, sem0, sem1):
    n_ids = ids_hbm.shape[0] // _NUM_CAT
    nch = n_ids // _CHUNK
    wid = lax.axis_index("s") * 2 + lax.axis_index("c")

    @pl.when(wid < _NUM_CAT)
    def _():
        def zero_body(j, carry):
            hist_v[pl.ds(j * _LANES, _LANES)] = jnp.zeros((_LANES,), jnp.int32)
            return carry

        lax.fori_loop(0, _VOCAB // _LANES, zero_body, 0, unroll=16)

        sems = (sem0, sem1)

        def dma(c, b):
            base = wid * n_ids + c * _CHUNK
            return pltpu.make_async_copy(
                ids_hbm.at[pl.ds(base, _CHUNK)], idbuf.at[b], sems[b]
            )

        def process(b):
            def vec_body(j, carry):
                v = idbuf[b, pl.ds(j * _LANES, _LANES)]
                cnt, last = plsc.scan_count(v)
                plsc.addupdate_scatter(hist_v, [v], cnt, mask=last)
                return carry

            lax.fori_loop(0, _CHUNK // _LANES, vec_body, 0, unroll=8)

        dma(0, 0).start()

        def outer(k, carry):
            c0 = 2 * k
            dma(c0 + 1, 1).start()
            dma(c0, 0).wait()
            process(0)
            dma(jnp.minimum(c0 + 2, nch - 1), 0).start()
            dma(c0 + 1, 1).wait()
            process(1)
            return carry

        lax.fori_loop(0, nch // 2, outer, 0)
        dma(nch - 1, 0).wait()

        pltpu.sync_copy(hist_v, hist_hbm.at[pl.ds(wid * _VOCAB, _VOCAB)])


def _sc_hist(ids_flat):
    mesh = plsc.VectorSubcoreMesh(
        core_axis_name="c", subcore_axis_name="s", num_cores=2, num_subcores=16
    )
    return pl.kernel(
        _sc_hist_body,
        out_type=jax.ShapeDtypeStruct((_NUM_CAT * _VOCAB,), jnp.int32),
        mesh=mesh,
        scratch_types=[
            pltpu.VMEM((2, _CHUNK), jnp.int32),
            pltpu.VMEM((_VOCAB,), jnp.int32),
            pltpu.SemaphoreType.DMA,
            pltpu.SemaphoreType.DMA,
        ],
        compiler_params=pltpu.CompilerParams(needs_layout_passes=False),
    )(ids_flat)


def _tc_reduce_body(tab_ref, hist_ref, out_ref, acc_ref, *, n_total):
    g = pl.program_id(0)

    @pl.when(g == 0)
    def _():
        acc_ref[...] = jnp.zeros_like(acc_ref)

    x = tab_ref[0]
    x2 = x * x
    h = hist_ref[0].astype(jnp.float32)
    a = lax.rsqrt(h / n_total + 1e-9)
    acc_ref[...] += jax.lax.dot_general(
        a, x2, (((1,), (0,)), ((), ())), preferred_element_type=jnp.float32
    )

    @pl.when(g == _GRID - 1)
    def _():
        out_ref[0, 0] = jnp.sum(acc_ref[...])


def _tc_reduce(tables, hist3, n_total):
    body = functools.partial(_tc_reduce_body, n_total=float(n_total))
    return pl.pallas_call(
        body,
        grid=(_GRID,),
        in_specs=[
            pl.BlockSpec((1, _BV, _EMB_DIM), lambda g: (g // _NB, g % _NB, 0)),
            pl.BlockSpec((1, 1, _BV), lambda g: (g, 0, 0)),
        ],
        out_specs=pl.BlockSpec(memory_space=pltpu.SMEM),
        out_shape=jax.ShapeDtypeStruct((1, 1), jnp.float32),
        scratch_shapes=[pltpu.VMEM((1, _EMB_DIM), jnp.float32)],
    )(tables, hist3)


def kernel(inputs, tables):
    b, t, c = inputs.shape
    n_total = b * t
    ids = (
        inputs[:, :, c - _NUM_CAT :]
        .astype(jnp.int32)
        .reshape(n_total, _NUM_CAT)
        .T.reshape(-1)
    )
    hist = _sc_hist(ids)
    hist3 = hist.reshape(_GRID, 1, _BV)
    total = _tc_reduce(tables, hist3, n_total)
    loss = (_LAMBDA / _VOCAB) * total[0, 0]
    return (inputs, loss)


# speedup vs baseline: 3.9896x; 3.4051x over previous
import functools

import jax
import jax.numpy as jnp
from jax import lax
from jax.experimental import pallas as pl
from jax.experimental.pallas import tpu as pltpu
from jax.experimental.pallas import tpu_sc as plsc

_VOCAB = 100000
_NUM_CAT = 26
_EMB_DIM = 32
_LAMBDA = 0.001

_CHUNK = 4096
_LANES = 16


def _tc_ids_body(in_ref, out_ref):
    x = in_ref[0]
    out_ref[...] = x.astype(jnp.int32).reshape(out_ref.shape)


def _tc_ids(inputs_t):
    t, c, b = inputs_t.shape
    return pl.pallas_call(
        _tc_ids_body,
        grid=(t,),
        in_specs=[
            pl.BlockSpec((1, c, b), lambda i: (i, 0, 0)),
        ],
        out_specs=pl.BlockSpec((c * b,), lambda i: (i,)),
        out_shape=jax.ShapeDtypeStruct((t * c * b,), jnp.int32),
    )(inputs_t)


def _sc_hist_body(ids_hbm, hist_hbm, idbuf, hist_v, sem0, sem1, *, n_t, n_c, n_b, n_reg):
    nch = n_t
    wid = lax.axis_index("s") * 2 + lax.axis_index("c")

    @pl.when(wid < _NUM_CAT)
    def _():
        def zero_body(j, carry):
            hist_v[pl.ds(j * _LANES, _LANES)] = jnp.zeros((_LANES,), jnp.int32)
            return carry

        lax.fori_loop(0, _VOCAB // _LANES, zero_body, 0, unroll=16)

        sems = (sem0, sem1)

        def dma(c, b):
            base = c * (n_c * n_b) + (n_reg + wid) * n_b
            return pltpu.make_async_copy(
                ids_hbm.at[pl.ds(base, n_b)], idbuf.at[b], sems[b]
            )

        def process(b):
            def vec_body(j, carry):
                v = idbuf[b, pl.ds(j * _LANES, _LANES)]
                cnt, last = plsc.scan_count(v)
                plsc.addupdate_scatter(hist_v, [v], cnt, mask=last)
                return carry

            lax.fori_loop(0, n_b // _LANES, vec_body, 0, unroll=8)

        dma(0, 0).start()

        def outer(k, carry):
            c0 = 2 * k
            dma(c0 + 1, 1).start()
            dma(c0, 0).wait()
            process(0)
            dma(jnp.minimum(c0 + 2, nch - 1), 0).start()
            dma(c0 + 1, 1).wait()
            process(1)
            return carry

        lax.fori_loop(0, nch // 2, outer, 0)
        dma(nch - 1, 0).wait()

        pltpu.sync_copy(hist_v, hist_hbm.at[wid, 0])


def _sc_hist(ids_flat, n_t, n_c, n_b, n_reg):
    mesh = plsc.VectorSubcoreMesh(
        core_axis_name="c", subcore_axis_name="s", num_cores=2, num_subcores=16
    )
    body = functools.partial(
        _sc_hist_body, n_t=n_t, n_c=n_c, n_b=n_b, n_reg=n_reg
    )
    return pl.kernel(
        body,
        out_type=jax.ShapeDtypeStruct((_NUM_CAT, 1, _VOCAB), jnp.int32),
        mesh=mesh,
        scratch_types=[
            pltpu.VMEM((2, n_b), jnp.int32),
            pltpu.VMEM((_VOCAB,), jnp.int32),
            pltpu.SemaphoreType.DMA,
            pltpu.SemaphoreType.DMA,
        ],
        compiler_params=pltpu.CompilerParams(needs_layout_passes=False),
    )(ids_flat)


_KR = 8
_KS = _EMB_DIM // _KR


def _tc_reduce_body(tab_ref, hist_ref, out_ref, wacc_ref, *, n_total):
    g = pl.program_id(0)
    k = pl.program_id(1)

    @pl.when((g == 0) & (k == 0))
    def _():
        out_ref[0, 0] = 0.0

    x = tab_ref[...]
    p = jnp.sum(x * x, axis=0, keepdims=True)

    @pl.when(k == 0)
    def _():
        wacc_ref[...] = p

    @pl.when(k > 0)
    def _():
        wacc_ref[...] += p

    @pl.when(k == _KS - 1)
    def _():
        h = hist_ref[0].astype(jnp.float32)
        a = lax.rsqrt(h / n_total + 1e-9)
        out_ref[0, 0] += jnp.sum(a * wacc_ref[...])


def _tc_reduce(tables2, hist, n_total):
    body = functools.partial(_tc_reduce_body, n_total=float(n_total))
    return pl.pallas_call(
        body,
        grid=(_NUM_CAT, _KS),
        in_specs=[
            pl.BlockSpec((_KR, _VOCAB), lambda g, k: (g * _KS + k, 0)),
            pl.BlockSpec((1, 1, _VOCAB), lambda g, k: (g, 0, 0)),
        ],
        out_specs=pl.BlockSpec(memory_space=pltpu.SMEM),
        out_shape=jax.ShapeDtypeStruct((1, 1), jnp.float32),
        scratch_shapes=[pltpu.VMEM((1, _VOCAB), jnp.float32)],
    )(tables2, hist)


def kernel(inputs, tables):
    b, t, c = inputs.shape
    n_total = b * t
    inputs_t = lax.transpose(inputs, (1, 2, 0))
    tables2 = lax.transpose(tables, (0, 2, 1)).reshape(_NUM_CAT * _EMB_DIM, _VOCAB)
    ids = _tc_ids(inputs_t)
    hist = _sc_hist(ids, t, c, b, c - _NUM_CAT)
    total = _tc_reduce(tables2, hist, n_total)
    loss = (_LAMBDA / _VOCAB) * total[0, 0]
    return (inputs, loss)


# speedup vs baseline: 5.9587x; 1.4935x over previous
import functools

import jax
import jax.numpy as jnp
from jax import lax
from jax.experimental import pallas as pl
from jax.experimental.pallas import tpu as pltpu
from jax.experimental.pallas import tpu_sc as plsc

_VOCAB = 100000
_NUM_CAT = 26
_EMB_DIM = 32
_LAMBDA = 0.001

_CHUNK = 4096
_LANES = 16


def _tc_ids_body(in_ref, out_ref):
    x = in_ref[0]
    out_ref[...] = x.astype(jnp.int32).reshape(out_ref.shape)


def _tc_ids(inputs_t):
    t, c, b = inputs_t.shape
    return pl.pallas_call(
        _tc_ids_body,
        grid=(t,),
        in_specs=[
            pl.BlockSpec((1, c, b), lambda i: (i, 0, 0)),
        ],
        out_specs=pl.BlockSpec((c * b,), lambda i: (i,)),
        out_shape=jax.ShapeDtypeStruct((t * c * b,), jnp.int32),
    )(inputs_t)


def _sc_hist_body(ids_hbm, hist_hbm, idbuf, hist_v, sem0, sem1, *, n_t, n_c, n_b, n_reg):
    nch = n_t
    wid = lax.axis_index("s") * 2 + lax.axis_index("c")

    @pl.when(wid < _NUM_CAT)
    def _():
        def zero_body(j, carry):
            hist_v[pl.ds(j * _LANES, _LANES)] = jnp.zeros((_LANES,), jnp.int32)
            return carry

        lax.fori_loop(0, _VOCAB // _LANES, zero_body, 0, unroll=16)

        sems = (sem0, sem1)

        def dma(c, b):
            base = c * (n_c * n_b) + (n_reg + wid) * n_b
            return pltpu.make_async_copy(
                ids_hbm.at[pl.ds(base, n_b)], idbuf.at[b], sems[b]
            )

        def process(b):
            def vec_body(j, carry):
                v = idbuf[b, pl.ds(j * _LANES, _LANES)]
                cnt, last = plsc.scan_count(v)
                plsc.addupdate_scatter(hist_v, [v], cnt, mask=last)
                return carry

            lax.fori_loop(0, n_b // _LANES, vec_body, 0, unroll=8)

        dma(0, 0).start()

        def outer(k, carry):
            c0 = 2 * k
            dma(c0 + 1, 1).start()
            dma(c0, 0).wait()
            process(0)
            dma(jnp.minimum(c0 + 2, nch - 1), 0).start()
            dma(c0 + 1, 1).wait()
            process(1)
            return carry

        lax.fori_loop(0, nch // 2, outer, 0)
        dma(nch - 1, 0).wait()

        pltpu.sync_copy(hist_v, hist_hbm.at[wid, 0])


def _sc_hist(ids_flat, n_t, n_c, n_b, n_reg):
    mesh = plsc.VectorSubcoreMesh(
        core_axis_name="c", subcore_axis_name="s", num_cores=2, num_subcores=16
    )
    body = functools.partial(
        _sc_hist_body, n_t=n_t, n_c=n_c, n_b=n_b, n_reg=n_reg
    )
    return pl.kernel(
        body,
        out_type=jax.ShapeDtypeStruct((_NUM_CAT, 1, _VOCAB), jnp.int32),
        mesh=mesh,
        scratch_types=[
            pltpu.VMEM((2, n_b), jnp.int32),
            pltpu.VMEM((_VOCAB,), jnp.int32),
            pltpu.SemaphoreType.DMA,
            pltpu.SemaphoreType.DMA,
        ],
        compiler_params=pltpu.CompilerParams(needs_layout_passes=False),
    )(ids_flat)


_KR = 8
_KS = _EMB_DIM // _KR


def _tc_w2_body(tab_ref, out_ref):
    k = pl.program_id(1)
    x = tab_ref[...]
    p = jnp.sum(x * x, axis=0, keepdims=True)[None]

    @pl.when(k == 0)
    def _():
        out_ref[...] = p

    @pl.when(k > 0)
    def _():
        out_ref[...] += p


def _tc_w2(tables2):
    return pl.pallas_call(
        _tc_w2_body,
        grid=(_NUM_CAT, _KS),
        in_specs=[
            pl.BlockSpec((_KR, _VOCAB), lambda g, k: (g * _KS + k, 0)),
        ],
        out_specs=pl.BlockSpec((1, 1, _VOCAB), lambda g, k: (g, 0, 0)),
        out_shape=jax.ShapeDtypeStruct((_NUM_CAT, 1, _VOCAB), jnp.float32),
    )(tables2)


def _tc_combine_body(w2_ref, hist_ref, out_ref, *, n_total):
    g = pl.program_id(0)

    @pl.when(g == 0)
    def _():
        out_ref[0, 0] = 0.0

    h = hist_ref[0].astype(jnp.float32)
    a = lax.rsqrt(h / n_total + 1e-9)
    out_ref[0, 0] += jnp.sum(a * w2_ref[0])


def _tc_combine(w2, hist, n_total):
    body = functools.partial(_tc_combine_body, n_total=float(n_total))
    return pl.pallas_call(
        body,
        grid=(_NUM_CAT,),
        in_specs=[
            pl.BlockSpec((1, 1, _VOCAB), lambda g: (g, 0, 0)),
            pl.BlockSpec((1, 1, _VOCAB), lambda g: (g, 0, 0)),
        ],
        out_specs=pl.BlockSpec(memory_space=pltpu.SMEM),
        out_shape=jax.ShapeDtypeStruct((1, 1), jnp.float32),
    )(w2, hist)


def kernel(inputs, tables):
    b, t, c = inputs.shape
    n_total = b * t
    inputs_t = lax.transpose(inputs, (1, 2, 0))
    tables2 = lax.transpose(tables, (0, 2, 1)).reshape(_NUM_CAT * _EMB_DIM, _VOCAB)
    ids = _tc_ids(inputs_t)
    hist = _sc_hist(ids, t, c, b, c - _NUM_CAT)
    w2 = _tc_w2(tables2)
    total = _tc_combine(w2, hist, n_total)
    loss = (_LAMBDA / _VOCAB) * total[0, 0]
    return (inputs, loss)


# speedup vs baseline: 6.9060x; 1.1590x over previous
import functools

import jax
import jax.numpy as jnp
from jax import lax
from jax.experimental import pallas as pl
from jax.experimental.pallas import tpu as pltpu
from jax.experimental.pallas import tpu_sc as plsc

_VOCAB = 100000
_NUM_CAT = 26
_EMB_DIM = 32
_LAMBDA = 0.001

_CHUNK = 4096
_LANES = 16


def _tc_ids_body(in_ref, out_ref):
    x = in_ref[0]
    out_ref[...] = x.astype(jnp.int32).reshape(out_ref.shape)


def _tc_ids(inputs_t):
    t, c, b = inputs_t.shape
    return pl.pallas_call(
        _tc_ids_body,
        grid=(t,),
        in_specs=[
            pl.BlockSpec((1, c, b), lambda i: (i, 0, 0)),
        ],
        out_specs=pl.BlockSpec((c * b,), lambda i: (i,)),
        out_shape=jax.ShapeDtypeStruct((t * c * b,), jnp.int32),
    )(inputs_t)


def _sc_hist_body(ids_hbm, hist_hbm, idbuf, hist_v, sem0, sem1, *, n_t, n_c, n_b, n_reg):
    nch = n_t
    wid = lax.axis_index("s") * 2 + lax.axis_index("c")

    @pl.when(wid < _NUM_CAT)
    def _():
        def zero_body(j, carry):
            hist_v[pl.ds(j * _LANES, _LANES)] = jnp.zeros((_LANES,), jnp.int32)
            return carry

        lax.fori_loop(0, _VOCAB // _LANES, zero_body, 0, unroll=16)

        sems = (sem0, sem1)

        def dma(c, b):
            base = c * (n_c * n_b) + (n_reg + wid) * n_b
            return pltpu.make_async_copy(
                ids_hbm.at[pl.ds(base, n_b)], idbuf.at[b], sems[b]
            )

        ones = jnp.ones((_LANES,), jnp.int32)

        def process(b):
            def vec_body(j, carry):
                v = idbuf[b, pl.ds(j * _LANES, _LANES)]
                plsc.addupdate_scatter(hist_v, [v], ones)
                return carry

            lax.fori_loop(0, n_b // _LANES, vec_body, 0, unroll=8)

        dma(0, 0).start()

        def outer(k, carry):
            c0 = 2 * k
            dma(c0 + 1, 1).start()
            dma(c0, 0).wait()
            process(0)
            dma(jnp.minimum(c0 + 2, nch - 1), 0).start()
            dma(c0 + 1, 1).wait()
            process(1)
            return carry

        lax.fori_loop(0, nch // 2, outer, 0)
        dma(nch - 1, 0).wait()

        pltpu.sync_copy(hist_v, hist_hbm.at[wid, 0])


def _sc_hist(ids_flat, n_t, n_c, n_b, n_reg):
    mesh = plsc.VectorSubcoreMesh(
        core_axis_name="c", subcore_axis_name="s", num_cores=2, num_subcores=16
    )
    body = functools.partial(
        _sc_hist_body, n_t=n_t, n_c=n_c, n_b=n_b, n_reg=n_reg
    )
    return pl.kernel(
        body,
        out_type=jax.ShapeDtypeStruct((_NUM_CAT, 1, _VOCAB), jnp.int32),
        mesh=mesh,
        scratch_types=[
            pltpu.VMEM((2, n_b), jnp.int32),
            pltpu.VMEM((_VOCAB,), jnp.int32),
            pltpu.SemaphoreType.DMA,
            pltpu.SemaphoreType.DMA,
        ],
        compiler_params=pltpu.CompilerParams(needs_layout_passes=False),
    )(ids_flat)


_KR = 8
_KS = _EMB_DIM // _KR


def _tc_w2_body(tab_ref, out_ref):
    k = pl.program_id(1)
    x = tab_ref[...]
    p = jnp.sum(x * x, axis=0, keepdims=True)[None]

    @pl.when(k == 0)
    def _():
        out_ref[...] = p

    @pl.when(k > 0)
    def _():
        out_ref[...] += p


def _tc_w2(tables2):
    return pl.pallas_call(
        _tc_w2_body,
        grid=(_NUM_CAT, _KS),
        in_specs=[
            pl.BlockSpec((_KR, _VOCAB), lambda g, k: (g * _KS + k, 0)),
        ],
        out_specs=pl.BlockSpec((1, 1, _VOCAB), lambda g, k: (g, 0, 0)),
        out_shape=jax.ShapeDtypeStruct((_NUM_CAT, 1, _VOCAB), jnp.float32),
    )(tables2)


def _tc_combine_body(w2_ref, hist_ref, out_ref, *, n_total):
    g = pl.program_id(0)

    @pl.when(g == 0)
    def _():
        out_ref[0, 0] = 0.0

    h = hist_ref[0].astype(jnp.float32)
    a = lax.rsqrt(h / n_total + 1e-9)
    out_ref[0, 0] += jnp.sum(a * w2_ref[0])


def _tc_combine(w2, hist, n_total):
    body = functools.partial(_tc_combine_body, n_total=float(n_total))
    return pl.pallas_call(
        body,
        grid=(_NUM_CAT,),
        in_specs=[
            pl.BlockSpec((1, 1, _VOCAB), lambda g: (g, 0, 0)),
            pl.BlockSpec((1, 1, _VOCAB), lambda g: (g, 0, 0)),
        ],
        out_specs=pl.BlockSpec(memory_space=pltpu.SMEM),
        out_shape=jax.ShapeDtypeStruct((1, 1), jnp.float32),
    )(w2, hist)


def kernel(inputs, tables):
    b, t, c = inputs.shape
    n_total = b * t
    inputs_t = lax.transpose(inputs, (1, 2, 0))
    tables2 = lax.transpose(tables, (0, 2, 1)).reshape(_NUM_CAT * _EMB_DIM, _VOCAB)
    ids = _tc_ids(inputs_t)
    hist = _sc_hist(ids, t, c, b, c - _NUM_CAT)
    w2 = _tc_w2(tables2)
    total = _tc_combine(w2, hist, n_total)
    loss = (_LAMBDA / _VOCAB) * total[0, 0]
    return (inputs, loss)


# speedup vs baseline: 6.9083x; 1.0003x over previous
import functools

import jax
import jax.numpy as jnp
from jax import lax
from jax.experimental import pallas as pl
from jax.experimental.pallas import tpu as pltpu
from jax.experimental.pallas import tpu_sc as plsc

_VOCAB = 100000
_NUM_CAT = 26
_EMB_DIM = 32
_LAMBDA = 0.001

_CHUNK = 4096
_LANES = 16


def _tc_ids_body(in_ref, out_ref):
    x = in_ref[0]
    out_ref[...] = x.astype(jnp.int32).reshape(out_ref.shape)


def _tc_ids(inputs_t):
    t, c, b = inputs_t.shape
    return pl.pallas_call(
        _tc_ids_body,
        grid=(t,),
        in_specs=[
            pl.BlockSpec((1, c, b), lambda i: (i, 0, 0)),
        ],
        out_specs=pl.BlockSpec((c * b,), lambda i: (i,)),
        out_shape=jax.ShapeDtypeStruct((t * c * b,), jnp.int32),
    )(inputs_t)


def _sc_hist_body(ids_hbm, hist_hbm, idbuf, hist_v, sem0, sem1, *, n_t, n_c, n_b, n_reg):
    nch = n_t
    wid = lax.axis_index("s") * 2 + lax.axis_index("c")

    @pl.when(wid < _NUM_CAT)
    def _():
        @plsc.parallel_loop(0, _VOCAB // _LANES, unroll=16)
        def _(j):
            hist_v[pl.ds(j * _LANES, _LANES)] = jnp.zeros((_LANES,), jnp.int32)

        sems = (sem0, sem1)

        def dma(c, b):
            base = c * (n_c * n_b) + (n_reg + wid) * n_b
            return pltpu.make_async_copy(
                ids_hbm.at[pl.ds(base, n_b)], idbuf.at[b], sems[b]
            )

        def process(b):
            @plsc.parallel_loop(0, n_b // _LANES, unroll=8)
            def _(j):
                v = idbuf[b, pl.ds(j * _LANES, _LANES)]
                cnt, last = plsc.scan_count(v)
                plsc.addupdate_scatter(hist_v, [v], cnt, mask=last)

        dma(0, 0).start()

        def outer(k, carry):
            c0 = 2 * k
            dma(c0 + 1, 1).start()
            dma(c0, 0).wait()
            process(0)
            dma(jnp.minimum(c0 + 2, nch - 1), 0).start()
            dma(c0 + 1, 1).wait()
            process(1)
            return carry

        lax.fori_loop(0, nch // 2, outer, 0)
        dma(nch - 1, 0).wait()

        pltpu.sync_copy(hist_v, hist_hbm.at[wid, 0])


def _sc_hist(ids_flat, n_t, n_c, n_b, n_reg):
    mesh = plsc.VectorSubcoreMesh(
        core_axis_name="c", subcore_axis_name="s", num_cores=2, num_subcores=16
    )
    body = functools.partial(
        _sc_hist_body, n_t=n_t, n_c=n_c, n_b=n_b, n_reg=n_reg
    )
    return pl.kernel(
        body,
        out_type=jax.ShapeDtypeStruct((_NUM_CAT, 1, _VOCAB), jnp.int32),
        mesh=mesh,
        scratch_types=[
            pltpu.VMEM((2, n_b), jnp.int32),
            pltpu.VMEM((_VOCAB,), jnp.int32),
            pltpu.SemaphoreType.DMA,
            pltpu.SemaphoreType.DMA,
        ],
        compiler_params=pltpu.CompilerParams(needs_layout_passes=False),
    )(ids_flat)


_KR = 8
_KS = _EMB_DIM // _KR


def _tc_w2_body(tab_ref, out_ref):
    k = pl.program_id(1)
    x = tab_ref[...]
    p = jnp.sum(x * x, axis=0, keepdims=True)[None]

    @pl.when(k == 0)
    def _():
        out_ref[...] = p

    @pl.when(k > 0)
    def _():
        out_ref[...] += p


def _tc_w2(tables2):
    return pl.pallas_call(
        _tc_w2_body,
        grid=(_NUM_CAT, _KS),
        in_specs=[
            pl.BlockSpec((_KR, _VOCAB), lambda g, k: (g * _KS + k, 0)),
        ],
        out_specs=pl.BlockSpec((1, 1, _VOCAB), lambda g, k: (g, 0, 0)),
        out_shape=jax.ShapeDtypeStruct((_NUM_CAT, 1, _VOCAB), jnp.float32),
    )(tables2)


def _tc_combine_body(w2_ref, hist_ref, out_ref, *, n_total):
    g = pl.program_id(0)

    @pl.when(g == 0)
    def _():
        out_ref[0, 0] = 0.0

    h = hist_ref[0].astype(jnp.float32)
    a = lax.rsqrt(h / n_total + 1e-9)
    out_ref[0, 0] += jnp.sum(a * w2_ref[0])


def _tc_combine(w2, hist, n_total):
    body = functools.partial(_tc_combine_body, n_total=float(n_total))
    return pl.pallas_call(
        body,
        grid=(_NUM_CAT,),
        in_specs=[
            pl.BlockSpec((1, 1, _VOCAB), lambda g: (g, 0, 0)),
            pl.BlockSpec((1, 1, _VOCAB), lambda g: (g, 0, 0)),
        ],
        out_specs=pl.BlockSpec(memory_space=pltpu.SMEM),
        out_shape=jax.ShapeDtypeStruct((1, 1), jnp.float32),
    )(w2, hist)


def kernel(inputs, tables):
    b, t, c = inputs.shape
    n_total = b * t
    inputs_t = lax.transpose(inputs, (1, 2, 0))
    tables2 = lax.transpose(tables, (0, 2, 1)).reshape(_NUM_CAT * _EMB_DIM, _VOCAB)
    ids = _tc_ids(inputs_t)
    hist = _sc_hist(ids, t, c, b, c - _NUM_CAT)
    w2 = _tc_w2(tables2)
    total = _tc_combine(w2, hist, n_total)
    loss = (_LAMBDA / _VOCAB) * total[0, 0]
    return (inputs, loss)


# speedup vs baseline: 7.2458x; 1.0489x over previous
import functools

import jax
import jax.numpy as jnp
from jax import lax
from jax.experimental import pallas as pl
from jax.experimental.pallas import tpu as pltpu
from jax.experimental.pallas import tpu_sc as plsc

_VOCAB = 100000
_NUM_CAT = 26
_EMB_DIM = 32
_LAMBDA = 0.001

_CHUNK = 4096
_LANES = 16


def _tc_ids_body(in_ref, out_ref, *, n_reg):
    x = in_ref[0, n_reg:]
    out_ref[...] = x.astype(jnp.int32).reshape(out_ref.shape)


def _tc_ids(inputs_t):
    t, c, b = inputs_t.shape
    body = functools.partial(_tc_ids_body, n_reg=c - _NUM_CAT)
    return pl.pallas_call(
        body,
        grid=(t,),
        in_specs=[
            pl.BlockSpec((1, c, b), lambda i: (i, 0, 0)),
        ],
        out_specs=pl.BlockSpec((_NUM_CAT * b,), lambda i: (i,)),
        out_shape=jax.ShapeDtypeStruct((t * _NUM_CAT * b,), jnp.int32),
    )(inputs_t)


def _sc_hist_body(ids_hbm, hist_hbm, idbuf, hist_v, sem0, sem1, *, n_t, n_c, n_b, n_reg):
    nch = n_t
    wid = lax.axis_index("s") * 2 + lax.axis_index("c")

    @pl.when(wid < _NUM_CAT)
    def _():
        @plsc.parallel_loop(0, _VOCAB // _LANES, unroll=16)
        def _(j):
            hist_v[pl.ds(j * _LANES, _LANES)] = jnp.zeros((_LANES,), jnp.int32)

        sems = (sem0, sem1)

        def dma(c, b):
            base = c * (_NUM_CAT * n_b) + wid * n_b
            return pltpu.make_async_copy(
                ids_hbm.at[pl.ds(base, n_b)], idbuf.at[b], sems[b]
            )

        def process(b):
            @plsc.parallel_loop(0, n_b // _LANES, unroll=8)
            def _(j):
                v = idbuf[b, pl.ds(j * _LANES, _LANES)]
                cnt, last = plsc.scan_count(v)
                plsc.addupdate_scatter(hist_v, [v], cnt, mask=last)

        dma(0, 0).start()

        def outer(k, carry):
            c0 = 2 * k
            dma(c0 + 1, 1).start()
            dma(c0, 0).wait()
            process(0)
            dma(jnp.minimum(c0 + 2, nch - 1), 0).start()
            dma(c0 + 1, 1).wait()
            process(1)
            return carry

        lax.fori_loop(0, nch // 2, outer, 0)
        dma(nch - 1, 0).wait()

        pltpu.sync_copy(hist_v, hist_hbm.at[wid, 0])


def _sc_hist(ids_flat, n_t, n_c, n_b, n_reg):
    mesh = plsc.VectorSubcoreMesh(
        core_axis_name="c", subcore_axis_name="s", num_cores=2, num_subcores=16
    )
    body = functools.partial(
        _sc_hist_body, n_t=n_t, n_c=n_c, n_b=n_b, n_reg=n_reg
    )
    return pl.kernel(
        body,
        out_type=jax.ShapeDtypeStruct((_NUM_CAT, 1, _VOCAB), jnp.int32),
        mesh=mesh,
        scratch_types=[
            pltpu.VMEM((2, n_b), jnp.int32),
            pltpu.VMEM((_VOCAB,), jnp.int32),
            pltpu.SemaphoreType.DMA,
            pltpu.SemaphoreType.DMA,
        ],
        compiler_params=pltpu.CompilerParams(needs_layout_passes=False),
    )(ids_flat)


_KR = 8
_KS = _EMB_DIM // _KR


def _tc_w2_body(tab_ref, out_ref):
    k = pl.program_id(1)
    x = tab_ref[...]
    p = jnp.sum(x * x, axis=0, keepdims=True)[None]

    @pl.when(k == 0)
    def _():
        out_ref[...] = p

    @pl.when(k > 0)
    def _():
        out_ref[...] += p


def _tc_w2(tables2):
    return pl.pallas_call(
        _tc_w2_body,
        grid=(_NUM_CAT, _KS),
        in_specs=[
            pl.BlockSpec((_KR, _VOCAB), lambda g, k: (g * _KS + k, 0)),
        ],
        out_specs=pl.BlockSpec((1, 1, _VOCAB), lambda g, k: (g, 0, 0)),
        out_shape=jax.ShapeDtypeStruct((_NUM_CAT, 1, _VOCAB), jnp.float32),
    )(tables2)


_CC = 2


def _tc_combine_body(w2_ref, hist_ref, out_ref, *, n_total):
    g = pl.program_id(0)

    @pl.when(g == 0)
    def _():
        out_ref[0, 0] = 0.0

    h = hist_ref[:, 0].astype(jnp.float32)
    a = lax.rsqrt(h / n_total + 1e-9)
    out_ref[0, 0] += jnp.sum(a * w2_ref[:, 0])


def _tc_combine(w2, hist, n_total):
    body = functools.partial(_tc_combine_body, n_total=float(n_total))
    return pl.pallas_call(
        body,
        grid=(_NUM_CAT // _CC,),
        in_specs=[
            pl.BlockSpec((_CC, 1, _VOCAB), lambda g: (g, 0, 0)),
            pl.BlockSpec((_CC, 1, _VOCAB), lambda g: (g, 0, 0)),
        ],
        out_specs=pl.BlockSpec(memory_space=pltpu.SMEM),
        out_shape=jax.ShapeDtypeStruct((1, 1), jnp.float32),
    )(w2, hist)


def kernel(inputs, tables):
    b, t, c = inputs.shape
    n_total = b * t
    inputs_t = lax.transpose(inputs, (1, 2, 0))
    tables2 = lax.transpose(tables, (0, 2, 1)).reshape(_NUM_CAT * _EMB_DIM, _VOCAB)
    ids = _tc_ids(inputs_t)
    hist = _sc_hist(ids, t, c, b, c - _NUM_CAT)
    w2 = _tc_w2(tables2)
    total = _tc_combine(w2, hist, n_total)
    loss = (_LAMBDA / _VOCAB) * total[0, 0]
    return (inputs, loss)


# speedup vs baseline: 8.4529x; 1.1666x over previous
import functools

import jax
import jax.numpy as jnp
from jax import lax
from jax.experimental import pallas as pl
from jax.experimental.pallas import tpu as pltpu
from jax.experimental.pallas import tpu_sc as plsc

_VOCAB = 100000
_NUM_CAT = 26
_EMB_DIM = 32
_LAMBDA = 0.001

_CHUNK = 4096
_LANES = 16


def _tc_ids_body(in_ref, out_ref, *, n_reg):
    x = in_ref[0, n_reg:]
    out_ref[...] = x.astype(jnp.int32).reshape(out_ref.shape)


def _tc_ids(inputs_t):
    t, c, b = inputs_t.shape
    body = functools.partial(_tc_ids_body, n_reg=c - _NUM_CAT)
    return pl.pallas_call(
        body,
        grid=(t,),
        in_specs=[
            pl.BlockSpec((1, c, b), lambda i: (i, 0, 0)),
        ],
        out_specs=pl.BlockSpec((_NUM_CAT * b,), lambda i: (i,)),
        out_shape=jax.ShapeDtypeStruct((t * _NUM_CAT * b,), jnp.int32),
    )(inputs_t)


def _sc_hist_body(v5_hbm, hist_hbm, idbuf, hist_v, sem0, sem1, *, n_t, n_reg):
    nbt = v5_hbm.shape[2]
    wid = lax.axis_index("s") * 2 + lax.axis_index("c")

    @pl.when(wid < _NUM_CAT)
    def _():
        @plsc.parallel_loop(0, _VOCAB // _LANES, unroll=16)
        def _(j):
            hist_v[pl.ds(j * _LANES, _LANES)] = jnp.zeros((_LANES,), jnp.int32)

        sems = (sem0, sem1)
        col = n_reg + wid
        et = col // 8
        r = col % 8

        def dma(t, b):
            return pltpu.make_async_copy(
                v5_hbm.at[t, et, :, r], idbuf.at[b], sems[b]
            )

        def process(b):
            @plsc.parallel_loop(0, nbt * 8, unroll=8)
            def _(i):
                v = idbuf[b, i >> 3, pl.ds((i & 7) * _LANES, _LANES)]
                vi = v.astype(jnp.int32)
                cnt, last = plsc.scan_count(vi)
                plsc.addupdate_scatter(hist_v, [vi], cnt, mask=last)

        dma(0, 0).start()

        def outer(k, carry):
            t0 = 2 * k
            dma(t0 + 1, 1).start()
            dma(t0, 0).wait()
            process(0)
            dma(jnp.minimum(t0 + 2, n_t - 1), 0).start()
            dma(t0 + 1, 1).wait()
            process(1)
            return carry

        lax.fori_loop(0, n_t // 2, outer, 0)
        dma(n_t - 1, 0).wait()

        pltpu.sync_copy(hist_v, hist_hbm.at[wid, 0])


def _sc_hist(v5, n_t, n_reg):
    mesh = plsc.VectorSubcoreMesh(
        core_axis_name="c", subcore_axis_name="s", num_cores=2, num_subcores=16
    )
    body = functools.partial(_sc_hist_body, n_t=n_t, n_reg=n_reg)
    nbt = v5.shape[2]
    return pl.kernel(
        body,
        out_type=jax.ShapeDtypeStruct((_NUM_CAT, 1, _VOCAB), jnp.int32),
        mesh=mesh,
        scratch_types=[
            pltpu.VMEM((2, nbt, 128), jnp.float32),
            pltpu.VMEM((_VOCAB,), jnp.int32),
            pltpu.SemaphoreType.DMA,
            pltpu.SemaphoreType.DMA,
        ],
        compiler_params=pltpu.CompilerParams(needs_layout_passes=False),
    )(v5)


_KR = 8
_KS = _EMB_DIM // _KR


def _tc_w2_body(tab_ref, out_ref):
    k = pl.program_id(1)
    x = tab_ref[...]
    p = jnp.sum(x * x, axis=0, keepdims=True)[None]

    @pl.when(k == 0)
    def _():
        out_ref[...] = p

    @pl.when(k > 0)
    def _():
        out_ref[...] += p


def _tc_w2(tables2):
    return pl.pallas_call(
        _tc_w2_body,
        grid=(_NUM_CAT, _KS),
        in_specs=[
            pl.BlockSpec((_KR, _VOCAB), lambda g, k: (g * _KS + k, 0)),
        ],
        out_specs=pl.BlockSpec((1, 1, _VOCAB), lambda g, k: (g, 0, 0)),
        out_shape=jax.ShapeDtypeStruct((_NUM_CAT, 1, _VOCAB), jnp.float32),
    )(tables2)


_CC = 2


def _tc_combine_body(w2_ref, hist_ref, out_ref, *, n_total):
    g = pl.program_id(0)

    @pl.when(g == 0)
    def _():
        out_ref[0, 0] = 0.0

    h = hist_ref[:, 0].astype(jnp.float32)
    a = lax.rsqrt(h / n_total + 1e-9)
    out_ref[0, 0] += jnp.sum(a * w2_ref[:, 0])


def _tc_combine(w2, hist, n_total):
    body = functools.partial(_tc_combine_body, n_total=float(n_total))
    return pl.pallas_call(
        body,
        grid=(_NUM_CAT // _CC,),
        in_specs=[
            pl.BlockSpec((_CC, 1, _VOCAB), lambda g: (g, 0, 0)),
            pl.BlockSpec((_CC, 1, _VOCAB), lambda g: (g, 0, 0)),
        ],
        out_specs=pl.BlockSpec(memory_space=pltpu.SMEM),
        out_shape=jax.ShapeDtypeStruct((1, 1), jnp.float32),
    )(w2, hist)


def kernel(inputs, tables):
    b, t, c = inputs.shape
    n_total = b * t
    inputs_t = lax.transpose(inputs, (1, 2, 0))
    tables2 = lax.transpose(tables, (0, 2, 1)).reshape(_NUM_CAT * _EMB_DIM, _VOCAB)
    v5 = lax.transpose(
        inputs_t.reshape(t, c // 8, 8, b // 128, 128), (0, 1, 3, 2, 4)
    )
    hist = _sc_hist(v5, t, c - _NUM_CAT)
    w2 = _tc_w2(tables2)
    total = _tc_combine(w2, hist, n_total)
    loss = (_LAMBDA / _VOCAB) * total[0, 0]
    return (inputs, loss)


# speedup vs baseline: 10.5825x; 1.2519x over previous
import functools

import jax
import jax.numpy as jnp
from jax import lax
from jax.experimental import pallas as pl
from jax.experimental.pallas import tpu as pltpu
from jax.experimental.pallas import tpu_sc as plsc

_VOCAB = 100000
_NUM_CAT = 26
_EMB_DIM = 32
_LAMBDA = 0.001

_CHUNK = 4096
_LANES = 16


def _tc_ids_body(in_ref, out_ref, *, n_reg):
    x = in_ref[0, n_reg:]
    out_ref[...] = x.astype(jnp.int32).reshape(out_ref.shape)


def _tc_ids(inputs_t):
    t, c, b = inputs_t.shape
    body = functools.partial(_tc_ids_body, n_reg=c - _NUM_CAT)
    return pl.pallas_call(
        body,
        grid=(t,),
        in_specs=[
            pl.BlockSpec((1, c, b), lambda i: (i, 0, 0)),
        ],
        out_specs=pl.BlockSpec((_NUM_CAT * b,), lambda i: (i,)),
        out_shape=jax.ShapeDtypeStruct((t * _NUM_CAT * b,), jnp.int32),
    )(inputs_t)


def _sc_hist_body(v5_hbm, hist_hbm, idbuf, hist_v, sem0, sem1, *, n_t, n_reg):
    nbt = v5_hbm.shape[2]
    wid = lax.axis_index("s") * 2 + lax.axis_index("c")

    @pl.when(wid < _NUM_CAT)
    def _():
        @plsc.parallel_loop(0, _VOCAB // _LANES, unroll=16)
        def _(j):
            hist_v[pl.ds(j * _LANES, _LANES)] = jnp.zeros((_LANES,), jnp.int32)

        sems = (sem0, sem1)
        col = n_reg + wid
        et = col // 8
        r = col % 8

        def dma(t, b):
            return pltpu.make_async_copy(
                v5_hbm.at[t, et, :, r], idbuf.at[b], sems[b]
            )

        def process(b):
            @plsc.parallel_loop(0, nbt * 8, unroll=8)
            def _(i):
                v = idbuf[b, i >> 3, pl.ds((i & 7) * _LANES, _LANES)]
                vi = v.astype(jnp.int32)
                cnt, last = plsc.scan_count(vi)
                plsc.addupdate_scatter(hist_v, [vi], cnt, mask=last)

        dma(0, 0).start()

        def outer(k, carry):
            t0 = 2 * k
            dma(t0 + 1, 1).start()
            dma(t0, 0).wait()
            process(0)
            dma(jnp.minimum(t0 + 2, n_t - 1), 0).start()
            dma(t0 + 1, 1).wait()
            process(1)
            return carry

        lax.fori_loop(0, n_t // 2, outer, 0)
        dma(n_t - 1, 0).wait()

        pltpu.sync_copy(hist_v, hist_hbm.at[wid, 0])


def _sc_hist(v5, n_t, n_reg):
    mesh = plsc.VectorSubcoreMesh(
        core_axis_name="c", subcore_axis_name="s", num_cores=2, num_subcores=16
    )
    body = functools.partial(_sc_hist_body, n_t=n_t, n_reg=n_reg)
    nbt = v5.shape[2]
    return pl.kernel(
        body,
        out_type=jax.ShapeDtypeStruct((_NUM_CAT, 1, _VOCAB), jnp.int32),
        mesh=mesh,
        scratch_types=[
            pltpu.VMEM((2, nbt, 128), jnp.float32),
            pltpu.VMEM((_VOCAB,), jnp.int32),
            pltpu.SemaphoreType.DMA,
            pltpu.SemaphoreType.DMA,
        ],
        compiler_params=pltpu.CompilerParams(needs_layout_passes=False),
    )(v5)


_KR = 16
_KS = _EMB_DIM // _KR


def _tc_w2_body(tab_ref, out_ref):
    k = pl.program_id(1)
    x = tab_ref[...]
    p = jnp.sum(x * x, axis=0, keepdims=True)[None]

    @pl.when(k == 0)
    def _():
        out_ref[...] = p

    @pl.when(k > 0)
    def _():
        out_ref[...] += p


def _tc_w2(tables2):
    return pl.pallas_call(
        _tc_w2_body,
        grid=(_NUM_CAT, _KS),
        in_specs=[
            pl.BlockSpec((_KR, _VOCAB), lambda g, k: (g * _KS + k, 0)),
        ],
        out_specs=pl.BlockSpec((1, 1, _VOCAB), lambda g, k: (g, 0, 0)),
        out_shape=jax.ShapeDtypeStruct((_NUM_CAT, 1, _VOCAB), jnp.float32),
    )(tables2)


_CC = 2


def _tc_combine_body(w2_ref, hist_ref, out_ref, *, n_total):
    g = pl.program_id(0)

    @pl.when(g == 0)
    def _():
        out_ref[0, 0] = 0.0

    h = hist_ref[:, 0].astype(jnp.float32)
    a = lax.rsqrt(h / n_total + 1e-9)
    out_ref[0, 0] += jnp.sum(a * w2_ref[:, 0])


def _tc_combine(w2, hist, n_total):
    body = functools.partial(_tc_combine_body, n_total=float(n_total))
    return pl.pallas_call(
        body,
        grid=(_NUM_CAT // _CC,),
        in_specs=[
            pl.BlockSpec((_CC, 1, _VOCAB), lambda g: (g, 0, 0)),
            pl.BlockSpec((_CC, 1, _VOCAB), lambda g: (g, 0, 0)),
        ],
        out_specs=pl.BlockSpec(memory_space=pltpu.SMEM),
        out_shape=jax.ShapeDtypeStruct((1, 1), jnp.float32),
    )(w2, hist)


def kernel(inputs, tables):
    b, t, c = inputs.shape
    n_total = b * t
    inputs_t = lax.transpose(inputs, (1, 2, 0))
    tables2 = lax.transpose(tables, (0, 2, 1)).reshape(_NUM_CAT * _EMB_DIM, _VOCAB)
    v5 = lax.transpose(
        inputs_t.reshape(t, c // 8, 8, b // 128, 128), (0, 1, 3, 2, 4)
    )
    hist = _sc_hist(v5, t, c - _NUM_CAT)
    w2 = _tc_w2(tables2)
    total = _tc_combine(w2, hist, n_total)
    loss = (_LAMBDA / _VOCAB) * total[0, 0]
    return (inputs, loss)


# speedup vs baseline: 11.0254x; 1.0419x over previous
import functools

import jax
import jax.numpy as jnp
from jax import lax
from jax.experimental import pallas as pl
from jax.experimental.pallas import tpu as pltpu
from jax.experimental.pallas import tpu_sc as plsc

_VOCAB = 100000
_NUM_CAT = 26
_EMB_DIM = 32
_LAMBDA = 0.001

_CHUNK = 4096
_LANES = 16


def _tc_ids_body(in_ref, out_ref, *, n_reg):
    x = in_ref[0, n_reg:]
    out_ref[...] = x.astype(jnp.int32).reshape(out_ref.shape)


def _tc_ids(inputs_t):
    t, c, b = inputs_t.shape
    body = functools.partial(_tc_ids_body, n_reg=c - _NUM_CAT)
    return pl.pallas_call(
        body,
        grid=(t,),
        in_specs=[
            pl.BlockSpec((1, c, b), lambda i: (i, 0, 0)),
        ],
        out_specs=pl.BlockSpec((_NUM_CAT * b,), lambda i: (i,)),
        out_shape=jax.ShapeDtypeStruct((t * _NUM_CAT * b,), jnp.int32),
    )(inputs_t)


def _sc_hist_body(v5_hbm, hist_hbm, idbuf, hist_v, sem0, sem1, *, n_t, n_reg):
    nbt = v5_hbm.shape[2]
    wid = lax.axis_index("s") * 2 + lax.axis_index("c")

    @pl.when(wid < _NUM_CAT)
    def _():
        @plsc.parallel_loop(0, _VOCAB // _LANES, unroll=16)
        def _(j):
            hist_v[pl.ds(j * _LANES, _LANES)] = jnp.zeros((_LANES,), jnp.int32)

        sems = (sem0, sem1)
        col = n_reg + wid
        et = col // 8
        r = col % 8

        def dma(t, b):
            return pltpu.make_async_copy(
                v5_hbm.at[t, et, :, r], idbuf.at[b], sems[b]
            )

        def process(b):
            @plsc.parallel_loop(0, nbt * 8, unroll=8)
            def _(i):
                v = idbuf[b, i >> 3, pl.ds((i & 7) * _LANES, _LANES)]
                vi = v.astype(jnp.int32)
                cnt, last = plsc.scan_count(vi)
                plsc.addupdate_scatter(hist_v, [vi], cnt, mask=last)

        dma(0, 0).start()

        def outer(k, carry):
            t0 = 2 * k
            dma(t0 + 1, 1).start()
            dma(t0, 0).wait()
            process(0)
            dma(jnp.minimum(t0 + 2, n_t - 1), 0).start()
            dma(t0 + 1, 1).wait()
            process(1)
            return carry

        lax.fori_loop(0, n_t // 2, outer, 0)
        dma(n_t - 1, 0).wait()

        pltpu.sync_copy(hist_v, hist_hbm.at[wid, 0])


def _sc_hist(v5, n_t, n_reg):
    mesh = plsc.VectorSubcoreMesh(
        core_axis_name="c", subcore_axis_name="s", num_cores=2, num_subcores=16
    )
    body = functools.partial(_sc_hist_body, n_t=n_t, n_reg=n_reg)
    nbt = v5.shape[2]
    return pl.kernel(
        body,
        out_type=jax.ShapeDtypeStruct((_NUM_CAT, 1, _VOCAB), jnp.int32),
        mesh=mesh,
        scratch_types=[
            pltpu.VMEM((2, nbt, 128), jnp.float32),
            pltpu.VMEM((_VOCAB,), jnp.int32),
            pltpu.SemaphoreType.DMA,
            pltpu.SemaphoreType.DMA,
        ],
        compiler_params=pltpu.CompilerParams(needs_layout_passes=False),
    )(v5)


_KR = 16
_KS = _EMB_DIM // _KR


def _tc_w2_body(tab_ref, out_ref):
    k = pl.program_id(1)
    x = tab_ref[...]
    p = jnp.sum(x * x, axis=0, keepdims=True)[None]

    @pl.when(k == 0)
    def _():
        out_ref[...] = p

    @pl.when(k > 0)
    def _():
        out_ref[...] += p


def _tc_w2(tables2):
    return pl.pallas_call(
        _tc_w2_body,
        grid=(_NUM_CAT, _KS),
        in_specs=[
            pl.BlockSpec((_KR, _VOCAB), lambda g, k: (g * _KS + k, 0)),
        ],
        out_specs=pl.BlockSpec((1, 1, _VOCAB), lambda g, k: (g, 0, 0)),
        out_shape=jax.ShapeDtypeStruct((_NUM_CAT, 1, _VOCAB), jnp.float32),
    )(tables2)


_CC = 13


def _tc_combine_body(w2_ref, hist_ref, out_ref, *, n_total):
    g = pl.program_id(0)

    @pl.when(g == 0)
    def _():
        out_ref[0, 0] = 0.0

    h = hist_ref[:, 0].astype(jnp.float32)
    a = lax.rsqrt(h / n_total + 1e-9)
    out_ref[0, 0] += jnp.sum(a * w2_ref[:, 0])


def _tc_combine(w2, hist, n_total):
    body = functools.partial(_tc_combine_body, n_total=float(n_total))
    return pl.pallas_call(
        body,
        grid=(_NUM_CAT // _CC,),
        in_specs=[
            pl.BlockSpec((_CC, 1, _VOCAB), lambda g: (g, 0, 0)),
            pl.BlockSpec((_CC, 1, _VOCAB), lambda g: (g, 0, 0)),
        ],
        out_specs=pl.BlockSpec(memory_space=pltpu.SMEM),
        out_shape=jax.ShapeDtypeStruct((1, 1), jnp.float32),
    )(w2, hist)


def kernel(inputs, tables):
    b, t, c = inputs.shape
    n_total = b * t
    inputs_t = lax.transpose(inputs, (1, 2, 0))
    tables2 = lax.transpose(tables, (0, 2, 1)).reshape(_NUM_CAT * _EMB_DIM, _VOCAB)
    v5 = lax.transpose(
        inputs_t.reshape(t, c // 8, 8, b // 128, 128), (0, 1, 3, 2, 4)
    )
    hist = _sc_hist(v5, t, c - _NUM_CAT)
    w2 = _tc_w2(tables2)
    total = _tc_combine(w2, hist, n_total)
    loss = (_LAMBDA / _VOCAB) * total[0, 0]
    return (inputs, loss)
